# baseline (speedup 1.0000x reference)
import functools

import jax
import jax.numpy as jnp
from jax import lax
from jax.experimental import pallas as pl
from jax.experimental.pallas import tpu as pltpu
from jax.experimental.pallas import tpu_sc as plsc

_VOCAB = 100000
_EMB = 64
_SEQ = 8192
_BATCH = 4
_TOT = _BATCH * _SEQ
_NC = 2
_NS = 16
_NW = _NC * _NS
_PER_W = _TOT // _NW
_CHUNK = 128
_NCH = _PER_W // _CHUNK
_LANES = 16
_NB = 3
_DEPTH = 2


def _emb_body(xg_hbm, xo_hbm, tok2_hbm, pos_hbm, out_hbm,
              idx_v, off_v, rows_v, stage_v, gsem, osem, psem):
    cid = lax.axis_index("c")
    sid = lax.axis_index("s")
    wid = sid * _NC + cid
    base = pl.multiple_of(wid * _PER_W, _PER_W)
    pos_base = pl.multiple_of(lax.rem(base, _SEQ), _PER_W)

    xrow = pl.multiple_of(wid * _NCH, _NCH)
    pltpu.sync_copy(xg_hbm.at[pl.ds(xrow, _NCH)], idx_v)
    pltpu.sync_copy(xo_hbm.at[pl.ds(xrow, _NCH)], off_v)

    row_ids = [lax.iota(jnp.int32, _LANES) + 16 * t for t in range(_CHUNK // _LANES)]

    gathers = {}
    pos_cps = {}
    outs = {}
    for j in range(-_DEPTH, _NCH):
        f = j + _DEPTH
        if 0 <= f < _NCH:
            if f - _NB >= 0:
                outs[f - _NB].wait()
            gathers[f] = pltpu.async_copy(
                tok2_hbm.at[idx_v.at[f]], rows_v.at[f % _NB], gsem)
            pos_cps[f] = pltpu.async_copy(
                pos_hbm.at[pl.ds(pl.multiple_of(pos_base + f * _CHUNK, _CHUNK), _CHUNK)],
                stage_v.at[f % _NB], psem)
        if j < 0:
            continue

        gathers[j].wait()
        pos_cps[j].wait()
        slot = j % _NB

        rows2 = rows_v.at[slot]
        stage2 = stage_v.at[slot]
        offs_vecs = [off_v[j, pl.ds(16 * t, _LANES)] for t in range(_CHUNK // _LANES)]

        def col_add(c, _):
            c_splat = jnp.zeros((_LANES,), jnp.int32) + c
            for t in range(_CHUNK // _LANES):
                colv = offs_vecs[t] + c
                v = plsc.load_gather(rows2, [row_ids[t], colv])
                plsc.addupdate_scatter(stage2, [row_ids[t], c_splat], v)
            return 0

        lax.fori_loop(0, _EMB, col_add, 0, unroll=2)

        outs[j] = pltpu.async_copy(
            stage_v.at[slot],
            out_hbm.at[pl.ds(pl.multiple_of(base + j * _CHUNK, _CHUNK), _CHUNK)], osem)

    for j in range(_NCH - _NB, _NCH):
        if j >= 0:
            outs[j].wait()


@jax.jit
def _emb(xg, xo, tok2, pos_table):
    mesh = plsc.VectorSubcoreMesh(core_axis_name="c", subcore_axis_name="s")
    run = functools.partial(
        pl.kernel,
        mesh=mesh,
        out_type=jax.ShapeDtypeStruct((_TOT, _EMB), jnp.float32),
        scratch_types=[
            pltpu.VMEM((_NCH, _CHUNK), jnp.int32),
            pltpu.VMEM((_NCH, _CHUNK), jnp.int32),
            pltpu.VMEM((_NB, _CHUNK, 2 * _EMB), jnp.float32),
            pltpu.VMEM((_NB, _CHUNK, _EMB), jnp.float32),
            pltpu.SemaphoreType.DMA,
            pltpu.SemaphoreType.DMA,
            pltpu.SemaphoreType.DMA,
        ],
        compiler_params=pltpu.CompilerParams(use_tc_tiling_on_sc=True, needs_layout_passes=False),
    )(_emb_body)
    return run(xg, xo, tok2, pos_table)


def kernel(x, token_table, pos_table):
    xi = x.astype(jnp.int32).reshape(_NW * _NCH, _CHUNK)
    xg = xi >> 1
    xo = (xi & 1) * _EMB
    tok2 = token_table.reshape(_VOCAB // 2, 2 * _EMB)
    out = _emb(xg, xo, tok2, pos_table)
    return out.reshape(_BATCH, _SEQ, _EMB)


# speedup vs baseline: 1.6006x; 1.6006x over previous
import functools

import jax
import jax.numpy as jnp
from jax import lax
from jax.experimental import pallas as pl
from jax.experimental.pallas import tpu as pltpu
from jax.experimental.pallas import tpu_sc as plsc

_VOCAB = 100000
_EMB = 64
_SEQ = 8192
_BATCH = 4
_TOT = _BATCH * _SEQ
_NC = 2
_NS = 16
_NW = _NC * _NS
_PER_W = _TOT // _NW
_CHUNK = 128
_NCH = _PER_W // _CHUNK
_LANES = 16
_NB = 3
_DEPTH = 2


def _emb_body(x_hbm, tok_hbm, pos_hbm, out_hbm,
              idx_v, rows_v, stage_v, gsem, osem, psem):
    cid = lax.axis_index("c")
    sid = lax.axis_index("s")
    wid = sid * _NC + cid
    base = pl.multiple_of(wid * _PER_W, _PER_W)
    pos_base = pl.multiple_of(lax.rem(base, _SEQ), _PER_W)

    pltpu.sync_copy(x_hbm.at[pl.ds(pl.multiple_of(wid * _NCH, _NCH), _NCH)], idx_v)

    gathers = {}
    pos_cps = {}
    outs = {}
    for j in range(-_DEPTH, _NCH):
        f = j + _DEPTH
        if 0 <= f < _NCH:
            if f - _NB >= 0:
                outs[f - _NB].wait()
            gathers[f] = pltpu.async_copy(
                tok_hbm.at[idx_v.at[f]], rows_v.at[f % _NB], gsem)
            pos_cps[f] = pltpu.async_copy(
                pos_hbm.at[pl.ds(pl.multiple_of(pos_base + f * _CHUNK, _CHUNK), _CHUNK)],
                stage_v.at[f % _NB], psem)
        if j < 0:
            continue

        gathers[j].wait()
        pos_cps[j].wait()
        slot = j % _NB

        def row_add(r, _):
            for g in range(_EMB // _LANES):
                sl = pl.ds(g * _LANES, _LANES)
                plsc.addupdate(stage_v.at[slot, r, sl], rows_v[slot, r, sl])
            return 0

        lax.fori_loop(0, _CHUNK, row_add, 0, unroll=4)

        outs[j] = pltpu.async_copy(
            stage_v.at[slot],
            out_hbm.at[pl.ds(pl.multiple_of(base + j * _CHUNK, _CHUNK), _CHUNK)],
            osem)

    for j in range(_NCH - _NB, _NCH):
        if j >= 0:
            outs[j].wait()


@jax.jit
def _emb(xi, token_table, pos_table):
    mesh = plsc.VectorSubcoreMesh(core_axis_name="c", subcore_axis_name="s")
    run = functools.partial(
        pl.kernel,
        mesh=mesh,
        out_type=jax.ShapeDtypeStruct((_TOT, _EMB), jnp.float32),
        scratch_types=[
            pltpu.VMEM((_NCH, _CHUNK), jnp.int32),
            pltpu.VMEM((_NB, _CHUNK, _EMB), jnp.float32),
            pltpu.VMEM((_NB, _CHUNK, _EMB), jnp.float32),
            pltpu.SemaphoreType.DMA,
            pltpu.SemaphoreType.DMA,
            pltpu.SemaphoreType.DMA,
        ],
        compiler_params=pltpu.CompilerParams(use_tc_tiling_on_sc=False),
    )(_emb_body)
    return run(xi, token_table, pos_table)


def kernel(x, token_table, pos_table):
    xi = x.astype(jnp.int32).reshape(_NW * _NCH, _CHUNK)
    out = _emb(xi, token_table, pos_table)
    return out.reshape(_BATCH, _SEQ, _EMB)
